# baseline (speedup 1.0000x reference)
import functools
import math

import jax
import jax.numpy as jnp
from jax import lax
from jax.experimental import pallas as pl
from jax.experimental.pallas import tpu as pltpu
from jax.experimental.pallas import tpu_sc as plsc

VOCAB = 1000000
DIMS = 64
MAX_TOK = 200
BATCH = 4096
LANES = 16

NC, NS = 2, 16
NW = NC * NS
ROWS_PER_W = BATCH * MAX_TOK // NW
CHUNK = 128
NCHUNK = ROWS_PER_W // CHUNK
VPR = DIMS // LANES


def _pe_doubled():
    position = jnp.arange(0, MAX_TOK, dtype=jnp.float32)[:, None]
    div_term = jnp.exp(
        jnp.arange(0, DIMS, 2, dtype=jnp.float32) * -(math.log(10000.0) / DIMS))
    pe = jnp.zeros((MAX_TOK, DIMS), dtype=jnp.float32)
    pe = pe.at[:, 0::2].set(jnp.sin(position * div_term))
    pe = pe.at[:, 1::2].set(jnp.cos(position * div_term))
    return jnp.concatenate([pe, pe], axis=0)


_mesh = plsc.VectorSubcoreMesh(core_axis_name="c", subcore_axis_name="s")


@functools.partial(
    pl.kernel,
    out_type=jax.ShapeDtypeStruct((NW, NCHUNK, CHUNK, DIMS), jnp.float32),
    mesh=_mesh,
    compiler_params=pltpu.CompilerParams(
        needs_layout_passes=False, use_tc_tiling_on_sc=False),
    scratch_types=[
        pltpu.VMEM((NCHUNK, CHUNK), jnp.int32),
        pltpu.VMEM((2 * MAX_TOK, DIMS), jnp.float32),
        pltpu.VMEM((CHUNK, DIMS), jnp.float32),
        pltpu.VMEM((CHUNK, DIMS), jnp.float32),
        pltpu.SemaphoreType.DMA,
        pltpu.SemaphoreType.DMA,
    ],
)
def _emb_lookup(x_hbm, pe_hbm, table_hbm, out_hbm,
                idx_v, pe_v, gbuf, obuf, gsem, osem):
    wid = lax.axis_index("s") * NC + lax.axis_index("c")
    pltpu.sync_copy(x_hbm.at[wid], idx_v)
    pltpu.sync_copy(pe_hbm, pe_v)

    def chunk_body(c, carry):
        po = (c * CHUNK) % MAX_TOK
        pltpu.async_copy(table_hbm.at[idx_v.at[c]], gbuf, gsem).wait()

        def row_body(r, carry2):
            for c4 in range(VPR):
                sl = pl.ds(c4 * LANES, LANES)
                obuf[r, sl] = gbuf[r, sl] + pe_v[po + r, sl]
            return carry2
        lax.fori_loop(0, CHUNK, row_body, 0)

        zmask = idx_v[c, pl.ds(0, LANES)] == 0
        for g in range(1, CHUNK // LANES):
            zmask = zmask | (idx_v[c, pl.ds(g * LANES, LANES)] == 0)

        n0 = plsc.all_reduce_population_count(zmask)[0]

        @pl.when(n0 > 0)
        def _fixup():
            for g in range(CHUNK // LANES):
                zf = jnp.where(
                    idx_v[c, pl.ds(g * LANES, LANES)] == 0,
                    jnp.float32(1.0), jnp.float32(0.0))
                for rr in range(LANES):
                    br = lax.broadcast(zf[rr], (LANES,))
                    r = g * LANES + rr
                    for c4 in range(VPR):
                        sl = pl.ds(c4 * LANES, LANES)
                        obuf[r, sl] = obuf[r, sl] - br * gbuf[r, sl]

        pltpu.async_copy(obuf, out_hbm.at[wid, c], osem).wait()
        return carry

    lax.fori_loop(0, NCHUNK, chunk_body, 0)


def kernel(x, table):
    x_r = x.reshape(NW, NCHUNK, CHUNK).astype(jnp.int32)
    out = _emb_lookup(x_r, _pe_doubled(), table)
    return out.reshape(BATCH, MAX_TOK, DIMS)


# speedup vs baseline: 1.1750x; 1.1750x over previous
import functools
import math

import jax
import jax.numpy as jnp
from jax import lax
from jax.experimental import pallas as pl
from jax.experimental.pallas import tpu as pltpu
from jax.experimental.pallas import tpu_sc as plsc

VOCAB = 1000000
DIMS = 64
MAX_TOK = 200
BATCH = 4096
LANES = 16

NC, NS = 2, 16
NW = NC * NS
ROWS_PER_W = BATCH * MAX_TOK // NW
CHUNK = 128
NCHUNK = ROWS_PER_W // CHUNK
VPR = DIMS // LANES


def _pe_doubled():
    position = jnp.arange(0, MAX_TOK, dtype=jnp.float32)[:, None]
    div_term = jnp.exp(
        jnp.arange(0, DIMS, 2, dtype=jnp.float32) * -(math.log(10000.0) / DIMS))
    pe = jnp.zeros((MAX_TOK, DIMS), dtype=jnp.float32)
    pe = pe.at[:, 0::2].set(jnp.sin(position * div_term))
    pe = pe.at[:, 1::2].set(jnp.cos(position * div_term))
    return jnp.concatenate([pe, pe], axis=0)


_mesh = plsc.VectorSubcoreMesh(core_axis_name="c", subcore_axis_name="s")


@functools.partial(
    pl.kernel,
    out_type=jax.ShapeDtypeStruct((NW, NCHUNK, CHUNK, DIMS), jnp.float32),
    mesh=_mesh,
    compiler_params=pltpu.CompilerParams(
        needs_layout_passes=False, use_tc_tiling_on_sc=False),
    scratch_types=[
        pltpu.VMEM((NCHUNK, CHUNK), jnp.int32),
        pltpu.VMEM((2 * MAX_TOK, DIMS), jnp.float32),
        pltpu.VMEM((CHUNK, DIMS), jnp.float32),
        pltpu.VMEM((CHUNK, DIMS), jnp.float32),
        pltpu.VMEM((CHUNK, DIMS), jnp.float32),
        pltpu.VMEM((CHUNK, DIMS), jnp.float32),
        pltpu.SemaphoreType.DMA,
        pltpu.SemaphoreType.DMA,
        pltpu.SemaphoreType.DMA,
        pltpu.SemaphoreType.DMA,
    ],
)
def _emb_lookup(x_hbm, pe_hbm, table_hbm, out_hbm,
                idx_v, pe_v, gbuf0, gbuf1, obuf0, obuf1,
                gsem0, gsem1, osem0, osem1):
    wid = lax.axis_index("s") * NC + lax.axis_index("c")
    gb, ob = (gbuf0, gbuf1), (obuf0, obuf1)
    gs, os_ = (gsem0, gsem1), (osem0, osem1)
    pltpu.sync_copy(x_hbm.at[wid], idx_v)
    pltpu.sync_copy(pe_hbm, pe_v)

    pltpu.async_copy(table_hbm.at[idx_v.at[0]], gb[0], gs[0])
    pltpu.async_copy(table_hbm.at[idx_v.at[1]], gb[1], gs[1])

    @pl.loop(0, NCHUNK, step=2)
    def _chunks(c0):
        for b in range(2):
            c = c0 + b
            po = (c * CHUNK) % MAX_TOK
            pltpu.make_async_copy(
                table_hbm.at[idx_v.at[c]], gb[b], gs[b]).wait()

            @pl.when(c0 >= 2)
            def _wait_out():
                pltpu.make_async_copy(
                    ob[b], out_hbm.at[wid, c - 2], os_[b]).wait()

            @pl.loop(0, CHUNK, unroll=8)
            def _rows(r):
                for c4 in range(VPR):
                    sl = pl.ds(c4 * LANES, LANES)
                    ob[b][r, sl] = gb[b][r, sl] + pe_v[po + r, sl]

            zmask = idx_v[c, pl.ds(0, LANES)] == 0
            for g in range(1, CHUNK // LANES):
                zmask = zmask | (idx_v[c, pl.ds(g * LANES, LANES)] == 0)
            n0 = plsc.all_reduce_population_count(zmask)[0]

            @pl.when(n0 > 0)
            def _fixup():
                for g in range(CHUNK // LANES):
                    zf = jnp.where(
                        idx_v[c, pl.ds(g * LANES, LANES)] == 0,
                        jnp.float32(1.0), jnp.float32(0.0))
                    for rr in range(LANES):
                        br = lax.broadcast(zf[rr], (LANES,))
                        r = g * LANES + rr
                        for c4 in range(VPR):
                            sl = pl.ds(c4 * LANES, LANES)
                            ob[b][r, sl] = ob[b][r, sl] - br * gb[b][r, sl]

            @pl.when(c0 + 2 < NCHUNK)
            def _next_gather():
                pltpu.async_copy(
                    table_hbm.at[idx_v.at[c + 2]], gb[b], gs[b])

            pltpu.async_copy(ob[b], out_hbm.at[wid, c], os_[b])

    for b in range(2):
        c = NCHUNK - 2 + b
        pltpu.make_async_copy(ob[b], out_hbm.at[wid, c], os_[b]).wait()


def kernel(x, table):
    x_r = x.reshape(NW, NCHUNK, CHUNK).astype(jnp.int32)
    out = _emb_lookup(x_r, _pe_doubled(), table)
    return out.reshape(BATCH, MAX_TOK, DIMS)
